# baseline (speedup 1.0000x reference)
import functools

import jax
import jax.numpy as jnp
import numpy as np
from jax import lax
from jax.experimental import pallas as pl
from jax.experimental.pallas import tpu as pltpu
from jax.experimental.pallas import tpu_sc as plsc

_LANES = 16
_CHUNK = 128
_NB0, _NB1 = 54, 26


def _linear(x, wt, b, scale):
    n, d = x.shape
    dw = d // 2
    bm = 512

    def mm(x_ref, wt_ref, b_ref, o_ref):
        y = (
            jnp.dot(x_ref[...], wt_ref[...], preferred_element_type=jnp.float32)
            + b_ref[...]
        ) * scale
        u = lax.bitcast_convert_type(y, jnp.int32)

        def rne16(v):
            rnd = jnp.bitwise_and(lax.shift_right_logical(v, 16), 1) + 0x7FFF
            return lax.shift_right_logical(v + rnd, 16)

        o_ref[...] = jnp.bitwise_or(
            rne16(u[:, :dw]), lax.shift_left(rne16(u[:, dw:]), 16)
        )

    n_out = -(-n // 256) * 256
    return pl.pallas_call(
        mm,
        grid=(n_out // bm,),
        in_specs=[
            pl.BlockSpec((bm, d), lambda i: (i, 0)),
            pl.BlockSpec((d, d), lambda i: (0, 0)),
            pl.BlockSpec((1, d), lambda i: (0, 0)),
        ],
        out_specs=pl.BlockSpec((bm, dw), lambda i: (i, 0)),
        out_shape=jax.ShapeDtypeStruct((n_out, dw), jnp.int32),
    )(x, wt, b)


def _tree_sum(vals):
    while len(vals) > 1:
        nxt = [vals[i] + vals[i + 1] for i in range(0, len(vals) - 1, 2)]
        if len(vals) % 2:
            nxt.append(vals[-1])
        vals = nxt
    return vals[0]


def _gcn_gather_elu(h, idx_rows, kdeg, d, nout):
    npad, dw = h.shape
    nc, ns = 2, 16
    blk = 8
    chunks = (blk * kdeg) // _CHUNK
    nbsum = npad // (blk * ns)
    nb0, nb1 = _NB0, _NB1
    assert nb0 + nb1 == nbsum and nb0 % 4 == 2 and nb1 % 4 == 2
    idxr_w = nb0 * chunks
    nph = 4
    nlim = nout // blk
    assert nout % blk == 0

    mesh = plsc.VectorSubcoreMesh(core_axis_name="c", subcore_axis_name="s")

    @functools.partial(
        pl.kernel,
        mesh=mesh,
        compiler_params=pltpu.CompilerParams(use_tc_tiling_on_sc=False),
        out_type=jax.ShapeDtypeStruct((nout, d), jnp.float32),
        scratch_types=[
            pltpu.VMEM((idxr_w, _CHUNK), jnp.int32),
            pltpu.VMEM((4, blk * kdeg, dw), jnp.int32),
            pltpu.VMEM((4, blk, dw), jnp.int32),
            pltpu.VMEM((4, blk, d), jnp.float32),
            pltpu.SemaphoreType.DMA,
            pltpu.SemaphoreType.DMA,
            pltpu.SemaphoreType.DMA,
            pltpu.SemaphoreType.DMA,
            pltpu.SemaphoreType.DMA,
            pltpu.SemaphoreType.DMA,
            pltpu.SemaphoreType.DMA,
            pltpu.SemaphoreType.DMA,
        ],
    )
    def k(h_hbm, idx_hbm, out_hbm, idxv, stg, accv, outv,
          g0s, g1s, g2s, g3s, o0s, o1s, o2s, o3s):
        cidx = lax.axis_index("c")
        sidx = lax.axis_index("s")
        gsem = (g0s, g1s, g2s, g3s)
        osem = (o0s, o1s, o2s, o3s)
        start_blk = jnp.where(cidx == 0, sidx * nb0, ns * nb0 + sidx * nb1)
        nblk_w = jnp.where(cidx == 0, nb0, nb1)
        pltpu.sync_copy(idx_hbm.at[pl.ds(start_blk * chunks, idxr_w)], idxv)

        def n_of(b):
            return (start_blk + b) * blk

        def gather_copies(b, ph):
            cps = [
                pltpu.make_async_copy(
                    h_hbm.at[pl.ds(n_of(b), blk)], accv.at[ph], gsem[ph]
                )
            ]
            for c in range(chunks):
                cps.append(
                    pltpu.make_async_copy(
                        h_hbm.at[idxv.at[b * chunks + c]],
                        stg.at[ph, pl.ds(c * _CHUNK, _CHUNK)],
                        gsem[ph],
                    )
                )
            return cps

        def out_copy(b, ph):
            return pltpu.make_async_copy(
                outv.at[ph], out_hbm.at[pl.ds(n_of(b), blk)], osem[ph]
            )

        def unpack2(w):
            lo = lax.bitcast_convert_type(jnp.left_shift(w, 16), jnp.float32)
            hi = lax.bitcast_convert_type(
                jnp.bitwise_and(w, jnp.int32(-65536)), jnp.float32
            )
            return lo, hi

        def reduce_block(ph):
            for j in range(blk):
                for g in range(d // (2 * _LANES)):
                    sl = pl.ds(g * _LANES, _LANES)
                    pairs = [unpack2(accv[ph, j, sl])] + [
                        unpack2(stg[ph, j * kdeg + kk, sl])
                        for kk in range(kdeg)
                    ]
                    va = _tree_sum([p[0] for p in pairs])
                    vb = _tree_sum([p[1] for p in pairs])
                    outv[ph, j, pl.ds(g * 2 * _LANES, _LANES)] = jnp.where(
                        va > 0.0, va, jnp.exp(va) - 1.0
                    )
                    outv[ph, j, pl.ds(g * 2 * _LANES + _LANES, _LANES)] = jnp.where(
                        vb > 0.0, vb, jnp.exp(vb) - 1.0
                    )

        def live(b):
            return start_blk + b < nlim

        for t in range(nph - 1):
            @pl.when(live(t))
            def _():
                for cp in gather_copies(t, t):
                    cp.start()

        def body(b, carry):
            ph = b & 3
            pnx = (b + 3) & 3
            for p in range(nph):
                @pl.when(jnp.logical_and(pnx == p,
                                         jnp.logical_and(b + 3 < nblk_w,
                                                         live(b + 3))))
                def _():
                    for cp in gather_copies(b + 3, p):
                        cp.start()

            for p in range(nph):
                @pl.when(jnp.logical_and(ph == p, live(b)))
                def _():
                    for cp in gather_copies(b, p):
                        cp.wait()

                    @pl.when(b >= 4)
                    def _():
                        out_copy(b - 4, p).wait()

            @pl.when(live(b))
            def _():
                reduce_block(ph)

            for p in range(nph):
                @pl.when(jnp.logical_and(ph == p, live(b)))
                def _():
                    out_copy(b, p).start()
            return carry

        lax.fori_loop(0, nblk_w, body, 0)
        nlive = jnp.minimum(nblk_w, jnp.maximum(nlim - start_blk, 0))
        for t in range(nph):
            bt = nlive - nph + t

            @pl.when(bt >= 0)
            def _():
                for p in range(nph):
                    @pl.when(jnp.bitwise_and(bt, 3) == p)
                    def _():
                        out_copy(bt, p).wait()

    return k(h, idx_rows)


def kernel(x, edge_index, W0, b0, W1, b1):
    n, d = x.shape
    kdeg = edge_index.shape[1]
    scale = 1.0 / (kdeg + 1)
    npad = -(-n // 256) * 256
    chunks = 8 * kdeg // _CHUNK
    ns = 16
    rows_needed = (ns * _NB0 + (ns - 1) * _NB1) * chunks + _NB0 * chunks
    idx_rows = edge_index.reshape(n * kdeg // _CHUNK, _CHUNK)
    idx_rows = jnp.concatenate(
        [
            idx_rows,
            jnp.zeros((rows_needed - n * kdeg // _CHUNK, _CHUNK), jnp.int32),
        ],
        axis=0,
    )

    arr = np.arange(d).reshape(d // 32, 2, 16)
    perm = np.concatenate([arr[:, 0, :].reshape(-1), arr[:, 1, :].reshape(-1)])

    h0 = _linear(x, W0.T[:, perm], b0[perm].reshape(1, d), scale)
    g0 = _gcn_gather_elu(h0, idx_rows, kdeg, d, n)
    h1 = _linear(g0, W1.T[:, perm], b1[perm].reshape(1, d), scale)
    g1 = _gcn_gather_elu(h1, idx_rows, kdeg, d, n)
    return g1


# speedup vs baseline: 1.1015x; 1.1015x over previous
import functools

import jax
import jax.numpy as jnp
import numpy as np
from jax import lax
from jax.experimental import pallas as pl
from jax.experimental.pallas import tpu as pltpu
from jax.experimental.pallas import tpu_sc as plsc

_LANES = 16
_CHUNK = 128
_NB0, _NB1 = 54, 26


def _linear(x, wt, b, scale):
    n, d = x.shape
    dw = d // 2
    bm = 512

    def mm(x_ref, wt_ref, b_ref, o_ref):
        y = (
            jnp.dot(x_ref[...], wt_ref[...], preferred_element_type=jnp.float32)
            + b_ref[...]
        ) * scale
        u = lax.bitcast_convert_type(y, jnp.int32)

        def rne16(v):
            rnd = jnp.bitwise_and(lax.shift_right_logical(v, 16), 1) + 0x7FFF
            return lax.shift_right_logical(v + rnd, 16)

        o_ref[...] = jnp.bitwise_or(
            rne16(u[:, :dw]), lax.shift_left(rne16(u[:, dw:]), 16)
        )

    n_out = -(-n // 256) * 256
    return pl.pallas_call(
        mm,
        grid=(n_out // bm,),
        in_specs=[
            pl.BlockSpec((bm, d), lambda i: (i, 0)),
            pl.BlockSpec((d, d), lambda i: (0, 0)),
            pl.BlockSpec((1, d), lambda i: (0, 0)),
        ],
        out_specs=pl.BlockSpec((bm, dw), lambda i: (i, 0)),
        out_shape=jax.ShapeDtypeStruct((n_out, dw), jnp.int32),
    )(x, wt, b)


def _tree_sum(vals):
    while len(vals) > 1:
        nxt = [vals[i] + vals[i + 1] for i in range(0, len(vals) - 1, 2)]
        if len(vals) % 2:
            nxt.append(vals[-1])
        vals = nxt
    return vals[0]


def _gcn_gather_elu(h, idx_rows, kdeg, d, nout):
    npad, dw = h.shape
    nc, ns = 2, 16
    blk = 8
    chunks = (blk * kdeg) // _CHUNK
    nbsum = npad // (blk * ns)
    nb0, nb1 = _NB0, _NB1
    assert nb0 + nb1 == nbsum and nb0 % 4 == 2 and nb1 % 4 == 2
    idxr_w = nb0 * chunks
    nph = 4
    nlim = nout // blk
    assert nout % blk == 0

    mesh = plsc.VectorSubcoreMesh(core_axis_name="c", subcore_axis_name="s")

    @functools.partial(
        pl.kernel,
        mesh=mesh,
        compiler_params=pltpu.CompilerParams(use_tc_tiling_on_sc=False),
        out_type=jax.ShapeDtypeStruct((nout, d), jnp.float32),
        scratch_types=[
            pltpu.VMEM((idxr_w, _CHUNK), jnp.int32),
            pltpu.VMEM((4, blk * kdeg, dw), jnp.int32),
            pltpu.VMEM((4, blk, dw), jnp.int32),
            pltpu.VMEM((4, blk, d), jnp.float32),
            pltpu.SemaphoreType.DMA,
            pltpu.SemaphoreType.DMA,
            pltpu.SemaphoreType.DMA,
            pltpu.SemaphoreType.DMA,
            pltpu.SemaphoreType.DMA,
            pltpu.SemaphoreType.DMA,
            pltpu.SemaphoreType.DMA,
            pltpu.SemaphoreType.DMA,
        ],
    )
    def k(h_hbm, idx_hbm, out_hbm, idxv, stg, accv, outv,
          g0s, g1s, g2s, g3s, o0s, o1s, o2s, o3s):
        cidx = lax.axis_index("c")
        sidx = lax.axis_index("s")
        gsem = (g0s, g1s, g2s, g3s)
        osem = (o0s, o1s, o2s, o3s)
        start_blk = jnp.where(cidx == 0, sidx * nb0, ns * nb0 + sidx * nb1)
        nblk_w = jnp.where(cidx == 0, nb0, nb1)
        pltpu.sync_copy(idx_hbm.at[pl.ds(start_blk * chunks, idxr_w)], idxv)

        def n_of(b):
            return (start_blk + b) * blk

        def gather_copies(b, ph):
            cps = [
                pltpu.make_async_copy(
                    h_hbm.at[pl.ds(n_of(b), blk)], accv.at[ph], gsem[ph]
                )
            ]
            for c in range(chunks):
                cps.append(
                    pltpu.make_async_copy(
                        h_hbm.at[idxv.at[b * chunks + c]],
                        stg.at[ph, pl.ds(c * _CHUNK, _CHUNK)],
                        gsem[ph],
                    )
                )
            return cps

        def out_copy(b, ph):
            return pltpu.make_async_copy(
                outv.at[ph], out_hbm.at[pl.ds(n_of(b), blk)], osem[ph]
            )

        def unpack2(w):
            lo = lax.bitcast_convert_type(jnp.left_shift(w, 16), jnp.float32)
            hi = lax.bitcast_convert_type(
                jnp.bitwise_and(w, jnp.int32(-65536)), jnp.float32
            )
            return lo, hi

        def reduce_block(ph):
            for j in range(blk):
                for g in range(d // (2 * _LANES)):
                    sl = pl.ds(g * _LANES, _LANES)
                    pairs = [unpack2(accv[ph, j, sl])] + [
                        unpack2(stg[ph, j * kdeg + kk, sl])
                        for kk in range(kdeg)
                    ]
                    va = _tree_sum([p[0] for p in pairs])
                    vb = _tree_sum([p[1] for p in pairs])
                    outv[ph, j, pl.ds(g * 2 * _LANES, _LANES)] = jnp.where(
                        va > 0.0, va, jnp.exp(va) - 1.0
                    )
                    outv[ph, j, pl.ds(g * 2 * _LANES + _LANES, _LANES)] = jnp.where(
                        vb > 0.0, vb, jnp.exp(vb) - 1.0
                    )

        def live(b):
            return start_blk + b < nlim

        for t in range(nph - 1):
            @pl.when(live(t))
            def _():
                for cp in gather_copies(t, t):
                    cp.start()

        def body(b, carry):
            ph = b & 3
            pnx = (b + 3) & 3
            for p in range(nph):
                @pl.when(jnp.logical_and(pnx == p,
                                         jnp.logical_and(b + 3 < nblk_w,
                                                         live(b + 3))))
                def _():
                    for cp in gather_copies(b + 3, p):
                        cp.start()

            for p in range(nph):
                @pl.when(jnp.logical_and(ph == p, live(b)))
                def _():
                    for cp in gather_copies(b, p):
                        cp.wait()

                    @pl.when(b >= 4)
                    def _():
                        out_copy(b - 4, p).wait()

                    reduce_block(p)
                    out_copy(b, p).start()
            return carry

        lax.fori_loop(0, nblk_w, body, 0)
        nlive = jnp.minimum(nblk_w, jnp.maximum(nlim - start_blk, 0))
        for t in range(nph):
            bt = nlive - nph + t

            @pl.when(bt >= 0)
            def _():
                for p in range(nph):
                    @pl.when(jnp.bitwise_and(bt, 3) == p)
                    def _():
                        out_copy(bt, p).wait()

    return k(h, idx_rows)


def kernel(x, edge_index, W0, b0, W1, b1):
    n, d = x.shape
    kdeg = edge_index.shape[1]
    scale = 1.0 / (kdeg + 1)
    npad = -(-n // 256) * 256
    chunks = 8 * kdeg // _CHUNK
    ns = 16
    rows_needed = (ns * _NB0 + (ns - 1) * _NB1) * chunks + _NB0 * chunks
    idx_rows = edge_index.reshape(n * kdeg // _CHUNK, _CHUNK)
    idx_rows = jnp.concatenate(
        [
            idx_rows,
            jnp.zeros((rows_needed - n * kdeg // _CHUNK, _CHUNK), jnp.int32),
        ],
        axis=0,
    )

    arr = np.arange(d).reshape(d // 32, 2, 16)
    perm = np.concatenate([arr[:, 0, :].reshape(-1), arr[:, 1, :].reshape(-1)])

    h0 = _linear(x, W0.T[:, perm], b0[perm].reshape(1, d), scale)
    g0 = _gcn_gather_elu(h0, idx_rows, kdeg, d, n)
    h1 = _linear(g0, W1.T[:, perm], b1[perm].reshape(1, d), scale)
    g1 = _gcn_gather_elu(h1, idx_rows, kdeg, d, n)
    return g1
